# baseline (speedup 1.0000x reference)
import functools

import jax
import jax.numpy as jnp
from jax import lax
from jax.experimental import pallas as pl
from jax.experimental.pallas import tpu as pltpu
from jax.experimental.pallas import tpu_sc as plsc

_NEG = float("-inf")
_BIG = 2**30


def _topk_body(logp3_ref, blsg_ref, topp_ref, q_ref, w_ref, gmax_ref):
    nblk, nj, nl = logp3_ref.shape
    nsb = nblk // 8

    def s1(c, _):
        slab8 = logp3_ref[pl.ds(c * 8, 8)]
        gmax_ref[pl.ds(c * 8, 8), :] = jnp.max(slab8, axis=1)
        return c + 1, None

    lax.scan(s1, 0, None, length=nsb)

    adj_all = gmax_ref[:] + blsg_ref[:]
    smax = jnp.max(adj_all.reshape(nsb, 8, nl), axis=1)

    sflat = lax.broadcasted_iota(jnp.int32, (nsb, nl), 0) * nl + \
        lax.broadcasted_iota(jnp.int32, (nsb, nl), 1)
    lane1 = lax.broadcasted_iota(jnp.int32, (1, nl), 1)
    jio = lax.broadcasted_iota(jnp.int32, (nj, nl), 0)
    lio = lax.broadcasted_iota(jnp.int32, (nj, nl), 1)
    io8 = lax.broadcasted_iota(jnp.int32, (8, nl), 0)
    lio8 = lax.broadcasted_iota(jnp.int32, (8, nl), 1)

    def step(t, carry):
        smax, topp, qa, wa = carry
        m_adj = jnp.max(smax)
        sl = jnp.min(jnp.where(smax == m_adj, sflat, _BIG))
        sstar = sl // nl
        lstar = sl % nl
        g8 = gmax_ref[pl.ds(sstar * 8, 8), :]
        b8 = blsg_ref[pl.ds(sstar * 8, 8), :]
        a8 = g8 + b8
        hit8 = (lio8 == lstar) & (a8 == m_adj)
        ioff = jnp.min(jnp.where(hit8, io8, _BIG))
        istar = sstar * 8 + ioff
        m_raw = jnp.max(jnp.where((io8 == ioff) & (lio8 == lstar), g8, _NEG))
        slab = logp3_ref[pl.ds(istar, 1)].reshape(nj, nl)
        gv = jnp.where(lio == lstar, slab, _NEG)
        jstar = jnp.min(jnp.where(gv == m_raw, jio, _BIG))
        nxt = jnp.max(jnp.where(gv < m_raw, gv, _NEG))
        g8u = jnp.where((io8 == ioff) & (lio8 == lstar), nxt, g8)
        gmax_ref[pl.ds(sstar * 8, 8), :] = g8u
        nsm = jnp.max(jnp.where(lio8 == lstar, g8u + b8, _NEG))
        smax = jnp.where(sflat == sl, nsm, smax)
        qstar = istar // 2
        vstar = (istar % 2) * 16384 + jstar * nl + lstar
        sel = lane1 == t
        topp = jnp.where(sel, m_adj, topp)
        qa = jnp.where(sel, qstar, qa)
        wa = jnp.where(sel, vstar, wa)
        return smax, topp, qa, wa

    init = (
        smax,
        jnp.full((1, nl), _NEG, jnp.float32),
        jnp.zeros((1, nl), jnp.int32),
        jnp.zeros((1, nl), jnp.int32),
    )
    _, topp, qa, wa = lax.fori_loop(0, 128, step, init)
    topp_ref[:] = topp
    q_ref[:] = qa
    w_ref[:] = wa


def _tc_topk(logp3, blsg):
    return pl.pallas_call(
        _topk_body,
        out_shape=[
            jax.ShapeDtypeStruct((1, 128), jnp.float32),
            jax.ShapeDtypeStruct((1, 128), jnp.int32),
            jax.ShapeDtypeStruct((1, 128), jnp.int32),
        ],
        scratch_shapes=[pltpu.VMEM((256, 128), jnp.float32)],
    )(logp3, blsg)


def _sc_gather_rows(table2, rr):
    info = plsc.get_sparse_core_info()
    nc = info.num_cores
    mesh = plsc.VectorSubcoreMesh(core_axis_name="c", subcore_axis_name="s")

    @functools.partial(
        pl.kernel,
        mesh=mesh,
        out_type=jax.ShapeDtypeStruct((128, 128), jnp.float32),
        scratch_types=[
            pltpu.VMEM((16,), jnp.int32),
            pltpu.VMEM((16, 128), jnp.float32),
            pltpu.SemaphoreType.DMA,
        ],
    )
    def gk(table_hbm, rr_hbm, out_hbm, idx_v, rows_v, sem):
        wid = lax.axis_index("s") * nc + lax.axis_index("c")

        @pl.when(wid < 8)
        def _():
            base = wid * 16
            pltpu.sync_copy(rr_hbm.at[pl.ds(base, 16)], idx_v)
            pltpu.async_copy(table_hbm.at[idx_v], rows_v, sem).wait()
            pltpu.sync_copy(rows_v, out_hbm.at[pl.ds(base, 16)])

    return gk(table2, rr)


def _lane_sel_body(rows_ref, llcol_ref, out_ref):
    lio = lax.broadcasted_iota(jnp.int32, (128, 128), 1)
    sel = lio == llcol_ref[:]
    out_ref[:] = jnp.max(
        jnp.where(sel, rows_ref[:], _NEG), axis=1, keepdims=True
    )


def _tc_lane_sel(rows, llcol):
    return pl.pallas_call(
        _lane_sel_body,
        out_shape=jax.ShapeDtypeStruct((128, 1), jnp.float32),
    )(rows, llcol)


def kernel(logprobsf, unaug_logprobsf, beam_logprobs_sum, k):
    b, v = logprobsf.shape
    logp3 = logprobsf.reshape(256, 128, 128)
    blsg = jnp.broadcast_to(
        jnp.repeat(beam_logprobs_sum, 2)[:, None], (256, 128)
    )
    topp, qa, wa = _tc_topk(logp3, blsg)
    top_p = topp.reshape(128)
    q_sel = qa.reshape(128)
    new_words = wa.reshape(128)
    rr = q_sel * (v // 128) + new_words // 128
    ll = new_words % 128
    rows = _sc_gather_rows(unaug_logprobsf.reshape(b * v // 128, 128), rr)
    new_r = _tc_lane_sel(rows, ll[:, None]).reshape(128)
    return new_words, top_p, new_r, q_sel


# speedup vs baseline: 1.2298x; 1.2298x over previous
import functools

import jax
import jax.numpy as jnp
from jax import lax
from jax.experimental import pallas as pl
from jax.experimental.pallas import tpu as pltpu
from jax.experimental.pallas import tpu_sc as plsc

_NEG = float("-inf")
_BIG = 2**30


def _topk_body(logp3_ref, blsg_ref, topp_ref, q_ref, w_ref, gmax_ref):
    nblk, nj, nl = logp3_ref.shape
    nsb = nblk // 8

    def s1(c, _):
        slab8 = logp3_ref[pl.ds(c * 8, 8)]
        gmax_ref[pl.ds(c * 8, 8), :] = jnp.max(slab8, axis=1)
        return c + 1, None

    lax.scan(s1, 0, None, length=nsb)

    adj0 = gmax_ref[:] + blsg_ref[:]

    ri = lax.broadcasted_iota(jnp.int32, (nblk, nl), 0)
    rl = lax.broadcasted_iota(jnp.int32, (nblk, nl), 1)
    rflat = ri * nl + rl
    lane1 = lax.broadcasted_iota(jnp.int32, (1, nl), 1)
    jio = lax.broadcasted_iota(jnp.int32, (nj, nl), 0)
    lio = lax.broadcasted_iota(jnp.int32, (nj, nl), 1)

    def step(t, carry):
        adj, topp, qa, wa = carry
        m_adj = jnp.max(adj)
        rstar = jnp.min(jnp.where(adj == m_adj, rflat, _BIG))
        istar = rstar // nl
        lstar = rstar % nl
        onl = lane1 == lstar
        row = gmax_ref[pl.ds(istar, 1), :]
        m_raw = jnp.max(jnp.where(onl, row, _NEG))
        brow = blsg_ref[pl.ds(istar, 1), :]
        bval = jnp.max(jnp.where(onl, brow, _NEG))
        slab = logp3_ref[pl.ds(istar, 1)].reshape(nj, nl)
        gv = jnp.where(lio == lstar, slab, _NEG)
        jstar = jnp.min(jnp.where(gv == m_raw, jio, _BIG))
        nxt = jnp.max(jnp.where(gv < m_raw, gv, _NEG))
        gmax_ref[pl.ds(istar, 1), :] = jnp.where(onl, nxt, row)
        adj = jnp.where(rflat == rstar, nxt + bval, adj)
        qstar = rstar // 256
        vstar = (istar % 2) * 16384 + jstar * nl + lstar
        sel = lane1 == t
        topp = jnp.where(sel, m_adj, topp)
        qa = jnp.where(sel, qstar, qa)
        wa = jnp.where(sel, vstar, wa)
        return adj, topp, qa, wa

    init = (
        adj0,
        jnp.full((1, nl), _NEG, jnp.float32),
        jnp.zeros((1, nl), jnp.int32),
        jnp.zeros((1, nl), jnp.int32),
    )
    _, topp, qa, wa = lax.fori_loop(0, 128, step, init)
    topp_ref[:] = topp
    q_ref[:] = qa
    w_ref[:] = wa


def _tc_topk(logp3, blsg):
    return pl.pallas_call(
        _topk_body,
        out_shape=[
            jax.ShapeDtypeStruct((1, 128), jnp.float32),
            jax.ShapeDtypeStruct((1, 128), jnp.int32),
            jax.ShapeDtypeStruct((1, 128), jnp.int32),
        ],
        scratch_shapes=[pltpu.VMEM((256, 128), jnp.float32)],
    )(logp3, blsg)


def _sc_gather_rows(table2, rr):
    info = plsc.get_sparse_core_info()
    nc = info.num_cores
    mesh = plsc.VectorSubcoreMesh(core_axis_name="c", subcore_axis_name="s")

    @functools.partial(
        pl.kernel,
        mesh=mesh,
        out_type=jax.ShapeDtypeStruct((128, 128), jnp.float32),
        scratch_types=[
            pltpu.VMEM((16,), jnp.int32),
            pltpu.VMEM((16, 128), jnp.float32),
            pltpu.SemaphoreType.DMA,
        ],
    )
    def gk(table_hbm, rr_hbm, out_hbm, idx_v, rows_v, sem):
        wid = lax.axis_index("s") * nc + lax.axis_index("c")

        @pl.when(wid < 8)
        def _():
            base = wid * 16
            pltpu.sync_copy(rr_hbm.at[pl.ds(base, 16)], idx_v)
            pltpu.async_copy(table_hbm.at[idx_v], rows_v, sem).wait()
            pltpu.sync_copy(rows_v, out_hbm.at[pl.ds(base, 16)])

    return gk(table2, rr)


def _lane_sel_body(rows_ref, llcol_ref, out_ref):
    lio = lax.broadcasted_iota(jnp.int32, (128, 128), 1)
    sel = lio == llcol_ref[:]
    out_ref[:] = jnp.max(
        jnp.where(sel, rows_ref[:], _NEG), axis=1, keepdims=True
    )


def _tc_lane_sel(rows, llcol):
    return pl.pallas_call(
        _lane_sel_body,
        out_shape=jax.ShapeDtypeStruct((128, 1), jnp.float32),
    )(rows, llcol)


def kernel(logprobsf, unaug_logprobsf, beam_logprobs_sum, k):
    b, v = logprobsf.shape
    logp3 = logprobsf.reshape(256, 128, 128)
    blsg = jnp.broadcast_to(
        jnp.repeat(beam_logprobs_sum, 2)[:, None], (256, 128)
    )
    topp, qa, wa = _tc_topk(logp3, blsg)
    top_p = topp.reshape(128)
    q_sel = qa.reshape(128)
    new_words = wa.reshape(128)
    rr = q_sel * (v // 128) + new_words // 128
    ll = new_words % 128
    rows = _sc_gather_rows(unaug_logprobsf.reshape(b * v // 128, 128), rr)
    new_r = _tc_lane_sel(rows, ll[:, None]).reshape(128)
    return new_words, top_p, new_r, q_sel
